# baseline (speedup 1.0000x reference)
import functools

import jax
import jax.numpy as jnp
from jax import lax
from jax.experimental import pallas as pl
from jax.experimental.pallas import tpu as pltpu
from jax.experimental.pallas import tpu_sc as plsc

_NC = 2
_NS = 16
_NW = _NC * _NS
_LANES = 16
_NEG = -3.0e38
_POS = 3.0e38
_BLK = 64


def _sc_mesh():
    return plsc.VectorSubcoreMesh(
        core_axis_name="c", subcore_axis_name="s", num_cores=_NC, num_subcores=_NS
    )


def _sc_params():
    return pltpu.CompilerParams(use_tc_tiling_on_sc=False, needs_layout_passes=False)


def _edge_scores_sc(xl, xr, ep, src, dst, att, np_):
    e_total, dout = ep.shape
    n = xl.shape[0]
    per_t = np_ // _NS
    per_w = e_total // _NW
    blk = _BLK
    nblk = per_w // blk
    nch = dout // _LANES
    ngrp = blk // _LANES

    @functools.partial(
        pl.kernel,
        out_type=(jax.ShapeDtypeStruct((e_total,), jnp.float32),
                  jax.ShapeDtypeStruct((2, np_), jnp.float32)),
        mesh=_sc_mesh(),
        compiler_params=_sc_params(),
        scratch_types=[
            pltpu.VMEM((blk,), jnp.int32),
            pltpu.VMEM((blk,), jnp.int32),
            pltpu.VMEM((blk,), jnp.int32),
            pltpu.VMEM((blk, dout), jnp.float32),
            pltpu.VMEM((blk, dout), jnp.float32),
            pltpu.VMEM((blk, dout), jnp.float32),
            pltpu.VMEM((dout,), jnp.float32),
            pltpu.VMEM((_LANES * _LANES,), jnp.float32),
            pltpu.VMEM((blk,), jnp.float32),
            pltpu.VMEM((np_,), jnp.float32),
            pltpu.VMEM((_NS, per_t), jnp.float32),
            pltpu.VMEM((per_t,), jnp.float32),
            pltpu.VMEM((_LANES,), jnp.int32),
            pltpu.VMEM((_LANES,), jnp.float32),
            pltpu.VMEM_SHARED((_NS, np_), jnp.float32),
            pltpu.SemaphoreType.DMA,
            pltpu.SemaphoreType.DMA,
        ],
    )
    def kern(xl_h, xr_h, ep_h, src_h, dst_h, att_h, e_h, m2_h,
             si, di, dic, rl, rr, re, attv, dots, ebuf, mloc, red, mout,
             ds_s, rm_s, stage_sh, sem1, sem2):
        cid = lax.axis_index("c")
        sid = lax.axis_index("s")
        wid = sid * _NC + cid
        pltpu.sync_copy(att_h, attv)
        lane = lax.iota(jnp.int32, _LANES)

        def init_body(r, _):
            mloc[pl.ds(r * _LANES, _LANES)] = jnp.full(
                (_LANES,), _NEG, jnp.float32)
            return 0

        lax.fori_loop(0, np_ // _LANES, init_body, 0)

        def blk_body(b, _):
            base = wid * per_w + b * blk
            pltpu.sync_copy(src_h.at[pl.ds(base, blk)], si)
            pltpu.sync_copy(dst_h.at[pl.ds(base, blk)], di)
            for g in range(ngrp):
                dic[pl.ds(g * _LANES, _LANES)] = jnp.minimum(
                    di[pl.ds(g * _LANES, _LANES)], n - 1)
            cl = pltpu.async_copy(xl_h.at[si], rl, sem1)
            cr = pltpu.async_copy(xr_h.at[dic], rr, sem2)
            pltpu.sync_copy(ep_h.at[pl.ds(base, blk)], re)
            cl.wait()
            cr.wait()

            def grp_body(g, _):
                for j in range(_LANES):
                    row = jnp.full((_LANES,), g * _LANES + j, jnp.int32)
                    acc = jnp.zeros((_LANES,), jnp.float32)
                    for c in range(nch):
                        col = c * _LANES + lane
                        s = (plsc.load_gather(rl, [row, col])
                             + plsc.load_gather(rr, [row, col])
                             + plsc.load_gather(re, [row, col]))
                        s = jnp.maximum(s, 0.2 * s)
                        acc = acc + s * attv[pl.ds(c * _LANES, _LANES)]
                    dots[pl.ds(j * _LANES, _LANES)] = acc
                e16 = jnp.zeros((_LANES,), jnp.float32)
                for t in range(_LANES):
                    e16 = e16 + plsc.load_gather(dots, [lane * _LANES + t])
                ebuf[pl.ds(g * _LANES, _LANES)] = e16

                dstg = plsc.load_gather(di, [g * _LANES + lane])
                ks, rm = plsc.sort_key_val(dstg, e16)
                ds_s[pl.ds(0, _LANES)] = ks
                rm_s[pl.ds(0, _LANES)] = rm
                for sh in (1, 2, 4, 8):
                    pidx = jnp.maximum(lane - sh, 0)
                    pd = plsc.load_gather(ds_s, [pidx])
                    pm = plsc.load_gather(rm_s, [pidx])
                    rm = jnp.where(pd == ks, jnp.maximum(rm, pm), rm)
                    rm_s[pl.ds(0, _LANES)] = rm
                nd = plsc.load_gather(ds_s, [jnp.minimum(lane + 1, _LANES - 1)])
                is_last = (nd != ks) | (lane == _LANES - 1)
                cur = plsc.load_gather(mloc, [ks])
                plsc.store_scatter(mloc, [ks], jnp.maximum(cur, rm),
                                   mask=is_last)
                return 0

            lax.fori_loop(0, ngrp, grp_body, 0)
            pltpu.sync_copy(ebuf, e_h.at[pl.ds(base, blk)])
            return 0

        lax.fori_loop(0, nblk, blk_body, 0)

        pltpu.sync_copy(mloc, stage_sh.at[sid])
        plsc.subcore_barrier()
        for r in range(_NS):
            pltpu.sync_copy(stage_sh.at[r, pl.ds(sid * per_t, per_t)],
                            red.at[r])
        for k in range(per_t // _LANES):
            v = red[0, pl.ds(k * _LANES, _LANES)]
            for r in range(1, _NS):
                v = jnp.maximum(v, red[r, pl.ds(k * _LANES, _LANES)])
            mout[pl.ds(k * _LANES, _LANES)] = v
        pltpu.sync_copy(mout, m2_h.at[cid, pl.ds(sid * per_t, per_t)])

    return kern(xl, xr, ep, src, dst, att)


def _aggregate_sc(xl, src, dst, e, m, q):
    e_total = src.shape[0]
    n, dout = xl.shape
    np_ = m.shape[0]
    nh = n // 4
    rpt = (nh + _NS - 1) // _NS
    nhp = rpt * _NS
    w_cols = dout + 16
    blk = 32
    nblk = e_total // _NS // blk
    nch = dout // _LANES
    ngrp = blk // _LANES

    @functools.partial(
        pl.kernel,
        out_type=jax.ShapeDtypeStruct((2, nhp, w_cols), jnp.float32),
        mesh=_sc_mesh(),
        compiler_params=_sc_params(),
        scratch_types=[
            pltpu.VMEM((np_,), jnp.float32),
            pltpu.VMEM((blk,), jnp.int32),
            pltpu.VMEM((blk,), jnp.int32),
            pltpu.VMEM((blk,), jnp.float32),
            pltpu.VMEM((blk,), jnp.float32),
            pltpu.VMEM((blk,), jnp.int32),
            pltpu.VMEM((blk,), jnp.int32),
            pltpu.VMEM((blk, dout), jnp.float32),
            pltpu.VMEM((blk, w_cols), jnp.float32),
            pltpu.VMEM((blk, w_cols), jnp.float32),
            pltpu.VMEM((8, w_cols), jnp.float32),
            pltpu.VMEM_SHARED((nhp, w_cols), jnp.float32),
            pltpu.SemaphoreType.DMA,
            pltpu.SemaphoreType.DMA,
            pltpu.SemaphoreType.DMA,
        ],
    )
    def kern(xl_h, src_h, dst_h, e_h, m_h, acc_h,
             mb, si, di, eb, wb, li_a, li_b, rows, stg_a, stg_b, zr, acc_sh,
             sem, sem_a, sem_b):
        cid = lax.axis_index("c")
        sid = lax.axis_index("s")
        lane = lax.iota(jnp.int32, _LANES)
        nbase = (2 * q + cid) * nh
        pltpu.sync_copy(m_h, mb)
        zv = jnp.zeros((_LANES,), jnp.float32)
        for r in range(8):
            rv = jnp.full((_LANES,), r, jnp.int32)
            for c in range(w_cols // _LANES):
                plsc.store_scatter(zr, [rv, c * _LANES + lane], zv)
        r0 = sid * rpt
        for z in range(rpt // 8):
            pltpu.sync_copy(zr, acc_sh.at[pl.ds(r0 + z * 8, 8)])
        if rpt % 8:
            pltpu.sync_copy(zr.at[pl.ds(0, rpt % 8)],
                            acc_sh.at[pl.ds(r0 + (rpt // 8) * 8, rpt % 8)])
        plsc.subcore_barrier()

        def half_blk(b2, k, stg, li, ssem):
            b = 2 * b2 + k
            base = sid * (e_total // _NS) + b * blk
            pltpu.sync_copy(src_h.at[pl.ds(base, blk)], si)
            pltpu.sync_copy(dst_h.at[pl.ds(base, blk)], di)
            pltpu.sync_copy(e_h.at[pl.ds(base, blk)], eb)
            cpy = pltpu.async_copy(xl_h.at[si], rows, sem)

            @pl.when(b2 > 0)
            def _():
                pltpu.make_async_copy(stg, acc_sh.at[li], ssem).wait()

            for g in range(ngrp):
                dstg = di[pl.ds(g * _LANES, _LANES)]
                eg = eb[pl.ds(g * _LANES, _LANES)]
                mg = plsc.load_gather(mb, [dstg])
                ex = jnp.exp(eg - mg)
                loc = dstg - nbase
                inr = (loc >= 0) & (loc < nh)
                wb[pl.ds(g * _LANES, _LANES)] = jnp.where(inr, ex, 0.0)
                li[pl.ds(g * _LANES, _LANES)] = jnp.clip(loc, 0, nhp - 1)
            cpy.wait()

            def edge_body(j, _):
                jv = jnp.full((_LANES,), j, jnp.int32)
                wv = plsc.load_gather(wb, [jv])
                for c in range(nch):
                    col = c * _LANES + lane
                    rowv = plsc.load_gather(rows, [jv, col])
                    plsc.store_scatter(stg, [jv, col], rowv * wv)
                plsc.store_scatter(stg, [jv, dout + lane],
                                   jnp.where(lane == 0, wv, 0.0))
                return 0

            lax.fori_loop(0, blk, edge_body, 0)
            pltpu.async_copy(stg, acc_sh.at[li], ssem, add=True)

        def blk_body(b2, _):
            half_blk(b2, 0, stg_a, li_a, sem_a)
            half_blk(b2, 1, stg_b, li_b, sem_b)
            return 0

        lax.fori_loop(0, nblk // 2, blk_body, 0)
        pltpu.make_async_copy(stg_a, acc_sh.at[li_a], sem_a).wait()
        pltpu.make_async_copy(stg_b, acc_sh.at[li_b], sem_b).wait()
        plsc.subcore_barrier()
        pltpu.sync_copy(acc_sh.at[pl.ds(sid * rpt, rpt)],
                        acc_h.at[cid, pl.ds(sid * rpt, rpt)])

    return kern(xl, src, dst, e, m)


def _gatv2_layer(x, src_p, dst_p, ea_p, loop_ea, p):
    n = x.shape[0]
    np_ = ((n + 16 * _NS - 1) // (16 * _NS)) * 16 * _NS
    xl = x @ p['Wl'] + p['bl']
    xr = x @ p['Wr'] + p['br']
    ep = ea_p @ p['We']
    lep = loop_ea @ p['We']

    e_edge, m2 = _edge_scores_sc(xl, xr, ep, src_p, dst_p, p['att'], np_)
    sf = xl + xr + lep
    sf = jnp.where(sf > 0, sf, 0.2 * sf)
    e_self = sf @ p['att']

    m_t = jnp.maximum(jnp.maximum(m2[0, :n], m2[1, :n]), e_self)
    ex_self = jnp.exp(e_self - m_t)
    m_pad = jnp.concatenate(
        [m_t, jnp.full((np_ - n,), _POS, jnp.float32)])

    dout = xl.shape[1]
    nh = n // 4
    acc0 = _aggregate_sc(xl, src_p, dst_p, e_edge, m_pad, 0)
    acc1 = _aggregate_sc(xl, src_p, dst_p, e_edge, m_pad, 1)
    num_e = jnp.concatenate(
        [acc0[0, :nh], acc0[1, :nh], acc1[0, :nh], acc1[1, :nh]], axis=0)
    den_e = num_e[:, dout]
    num = num_e[:, :dout] + ex_self[:, None] * xl
    den = den_e + ex_self
    out = num / (den[:, None] + 1e-16)
    return out + p['bias']


def _ln_relu_body(x_ref, g_ref, b_ref, o_ref, *, relu):
    x = x_ref[...]
    mu = jnp.mean(x, axis=-1, keepdims=True)
    var = jnp.mean((x - mu) ** 2, axis=-1, keepdims=True)
    y = (x - mu) * jax.lax.rsqrt(var + 1e-5) * g_ref[...] + b_ref[...]
    if relu:
        y = jnp.maximum(y, 0.0)
    o_ref[...] = y


def _ln_relu(x, g, b, relu):
    n, d = x.shape
    blk = 1000
    return pl.pallas_call(
        functools.partial(_ln_relu_body, relu=relu),
        grid=(n // blk,),
        in_specs=[
            pl.BlockSpec((blk, d), lambda i: (i, 0)),
            pl.BlockSpec((d,), lambda i: (0,)),
            pl.BlockSpec((d,), lambda i: (0,)),
        ],
        out_specs=pl.BlockSpec((blk, d), lambda i: (i, 0)),
        out_shape=jax.ShapeDtypeStruct((n, d), x.dtype),
    )(x, g, b)


def kernel(x, edge_index, edge_features, params):
    src, dst = edge_index[0], edge_index[1]
    e_real = src.shape[0]
    n = x.shape[0]
    per_w = -(-e_real // (_NW * _BLK)) * _BLK
    e_pad = per_w * _NW
    pad = e_pad - e_real
    src_p = jnp.concatenate([src, jnp.zeros((pad,), src.dtype)])
    dst_p = jnp.concatenate([dst, jnp.full((pad,), n, dst.dtype)])
    ea_p = jnp.concatenate(
        [edge_features, jnp.zeros((pad, edge_features.shape[1]),
                                  edge_features.dtype)], axis=0)
    cnt = jax.ops.segment_sum(jnp.ones((e_real,), dtype=x.dtype), dst,
                              num_segments=n)
    loop_ea = jax.ops.segment_sum(edge_features, dst, num_segments=n) \
        / jnp.clip(cnt, 1.0)[:, None]

    out = _gatv2_layer(x, src_p, dst_p, ea_p, loop_ea, params['conv1'])
    out = _ln_relu(out, params['conv1']['g'], params['conv1']['b'], True)
    out = _gatv2_layer(out, src_p, dst_p, ea_p, loop_ea, params['conv2'])
    out = _ln_relu(out, params['conv2']['g'], params['conv2']['b'], True)
    out = _gatv2_layer(out, src_p, dst_p, ea_p, loop_ea, params['conv3'])
    out = _ln_relu(out, params['conv3']['g'], params['conv3']['b'], False)
    return out


# speedup vs baseline: 1.5284x; 1.5284x over previous
import functools

import jax
import jax.numpy as jnp
from jax import lax
from jax.experimental import pallas as pl
from jax.experimental.pallas import tpu as pltpu
from jax.experimental.pallas import tpu_sc as plsc

_NC = 2
_NS = 16
_NW = _NC * _NS
_LANES = 16
_NEG = -3.0e38
_POS = 3.0e38
_BLK = 64


def _sc_mesh():
    return plsc.VectorSubcoreMesh(
        core_axis_name="c", subcore_axis_name="s", num_cores=_NC, num_subcores=_NS
    )


def _sc_params():
    return pltpu.CompilerParams(use_tc_tiling_on_sc=False, needs_layout_passes=False)


def _edge_scores_sc(xl, xr, ep, src, dst, att, np_):
    e_total, dout = ep.shape
    n = xl.shape[0]
    per_t = np_ // _NS
    per_w = e_total // _NW
    blk = _BLK
    nblk = per_w // blk
    nch = dout // _LANES
    ngrp = blk // _LANES

    @functools.partial(
        pl.kernel,
        out_type=(jax.ShapeDtypeStruct((e_total,), jnp.float32),
                  jax.ShapeDtypeStruct((2, np_), jnp.float32)),
        mesh=_sc_mesh(),
        compiler_params=_sc_params(),
        scratch_types=[
            pltpu.VMEM((blk,), jnp.int32),
            pltpu.VMEM((blk,), jnp.int32),
            pltpu.VMEM((blk,), jnp.int32),
            pltpu.VMEM((blk, dout), jnp.float32),
            pltpu.VMEM((blk, dout), jnp.float32),
            pltpu.VMEM((blk, dout), jnp.float32),
            pltpu.VMEM((dout,), jnp.float32),
            pltpu.VMEM((_LANES * _LANES,), jnp.float32),
            pltpu.VMEM((blk,), jnp.float32),
            pltpu.VMEM((np_,), jnp.float32),
            pltpu.VMEM((_NS, per_t), jnp.float32),
            pltpu.VMEM((per_t,), jnp.float32),
            pltpu.VMEM((_LANES,), jnp.int32),
            pltpu.VMEM((_LANES,), jnp.float32),
            pltpu.VMEM_SHARED((_NS, np_), jnp.float32),
            pltpu.SemaphoreType.DMA,
            pltpu.SemaphoreType.DMA,
        ],
    )
    def kern(xl_h, xr_h, ep_h, src_h, dst_h, att_h, e_h, m2_h,
             si, di, dic, rl, rr, re, attv, dots, ebuf, mloc, red, mout,
             ds_s, rm_s, stage_sh, sem1, sem2):
        cid = lax.axis_index("c")
        sid = lax.axis_index("s")
        wid = sid * _NC + cid
        pltpu.sync_copy(att_h, attv)
        lane = lax.iota(jnp.int32, _LANES)

        def init_body(r, _):
            mloc[pl.ds(r * _LANES, _LANES)] = jnp.full(
                (_LANES,), _NEG, jnp.float32)
            return 0

        lax.fori_loop(0, np_ // _LANES, init_body, 0)

        def blk_body(b, _):
            base = wid * per_w + b * blk
            pltpu.sync_copy(src_h.at[pl.ds(base, blk)], si)
            pltpu.sync_copy(dst_h.at[pl.ds(base, blk)], di)
            for g in range(ngrp):
                dic[pl.ds(g * _LANES, _LANES)] = jnp.minimum(
                    di[pl.ds(g * _LANES, _LANES)], n - 1)
            cl = pltpu.async_copy(xl_h.at[si], rl, sem1)
            cr = pltpu.async_copy(xr_h.at[dic], rr, sem2)
            pltpu.sync_copy(ep_h.at[pl.ds(base, blk)], re)
            cl.wait()
            cr.wait()

            def grp_body(g, _):
                for j in range(_LANES):
                    row = jnp.full((_LANES,), g * _LANES + j, jnp.int32)
                    acc = jnp.zeros((_LANES,), jnp.float32)
                    for c in range(nch):
                        col = c * _LANES + lane
                        s = (plsc.load_gather(rl, [row, col])
                             + plsc.load_gather(rr, [row, col])
                             + plsc.load_gather(re, [row, col]))
                        s = jnp.maximum(s, 0.2 * s)
                        acc = acc + s * attv[pl.ds(c * _LANES, _LANES)]
                    dots[pl.ds(j * _LANES, _LANES)] = acc
                e16 = jnp.zeros((_LANES,), jnp.float32)
                for t in range(_LANES):
                    e16 = e16 + plsc.load_gather(dots, [lane * _LANES + t])
                ebuf[pl.ds(g * _LANES, _LANES)] = e16

                dstg = plsc.load_gather(di, [g * _LANES + lane])
                ks, rm = plsc.sort_key_val(dstg, e16)
                ds_s[pl.ds(0, _LANES)] = ks
                rm_s[pl.ds(0, _LANES)] = rm
                for sh in (1, 2, 4, 8):
                    pidx = jnp.maximum(lane - sh, 0)
                    pd = plsc.load_gather(ds_s, [pidx])
                    pm = plsc.load_gather(rm_s, [pidx])
                    rm = jnp.where(pd == ks, jnp.maximum(rm, pm), rm)
                    rm_s[pl.ds(0, _LANES)] = rm
                nd = plsc.load_gather(ds_s, [jnp.minimum(lane + 1, _LANES - 1)])
                is_last = (nd != ks) | (lane == _LANES - 1)
                cur = plsc.load_gather(mloc, [ks])
                plsc.store_scatter(mloc, [ks], jnp.maximum(cur, rm),
                                   mask=is_last)
                return 0

            lax.fori_loop(0, ngrp, grp_body, 0)
            pltpu.sync_copy(ebuf, e_h.at[pl.ds(base, blk)])
            return 0

        lax.fori_loop(0, nblk, blk_body, 0)

        pltpu.sync_copy(mloc, stage_sh.at[sid])
        plsc.subcore_barrier()
        for r in range(_NS):
            pltpu.sync_copy(stage_sh.at[r, pl.ds(sid * per_t, per_t)],
                            red.at[r])
        for k in range(per_t // _LANES):
            v = red[0, pl.ds(k * _LANES, _LANES)]
            for r in range(1, _NS):
                v = jnp.maximum(v, red[r, pl.ds(k * _LANES, _LANES)])
            mout[pl.ds(k * _LANES, _LANES)] = v
        pltpu.sync_copy(mout, m2_h.at[cid, pl.ds(sid * per_t, per_t)])

    return kern(xl, xr, ep, src, dst, att)


def _aggregate_sc(xl, src, dst, e, m, q):
    e_total = src.shape[0]
    n, dout = xl.shape
    np_ = m.shape[0]
    nh = n // 4
    rpt = (nh + _NS - 1) // _NS
    nhp = rpt * _NS
    w_cols = dout + 16
    blk = 32
    nblk = e_total // _NS // blk
    nch = dout // _LANES
    ngrp = blk // _LANES

    @functools.partial(
        pl.kernel,
        out_type=jax.ShapeDtypeStruct((2, nhp, w_cols), jnp.float32),
        mesh=_sc_mesh(),
        compiler_params=_sc_params(),
        scratch_types=[
            pltpu.VMEM((np_,), jnp.float32),
            pltpu.VMEM((blk,), jnp.int32),
            pltpu.VMEM((blk,), jnp.int32),
            pltpu.VMEM((blk,), jnp.float32),
            pltpu.VMEM((blk,), jnp.float32),
            pltpu.VMEM((blk,), jnp.int32),
            pltpu.VMEM((blk,), jnp.int32),
            pltpu.VMEM((blk, dout), jnp.float32),
            pltpu.VMEM((blk, w_cols), jnp.float32),
            pltpu.VMEM((blk, w_cols), jnp.float32),
            pltpu.VMEM((8, w_cols), jnp.float32),
            pltpu.VMEM_SHARED((nhp, w_cols), jnp.float32),
            pltpu.SemaphoreType.DMA,
            pltpu.SemaphoreType.DMA,
            pltpu.SemaphoreType.DMA,
        ],
    )
    def kern(xl_h, src_h, dst_h, e_h, m_h, acc_h,
             mb, si, di, eb, wb, li_a, li_b, rows, stg_a, stg_b, zr, acc_sh,
             sem, sem_a, sem_b):
        cid = lax.axis_index("c")
        sid = lax.axis_index("s")
        lane = lax.iota(jnp.int32, _LANES)
        nbase = (2 * q + cid) * nh
        pltpu.sync_copy(m_h, mb)
        zv = jnp.zeros((_LANES,), jnp.float32)
        for r in range(8):
            rv = jnp.full((_LANES,), r, jnp.int32)
            for c in range(w_cols // _LANES):
                plsc.store_scatter(zr, [rv, c * _LANES + lane], zv)
        r0 = sid * rpt
        for z in range(rpt // 8):
            pltpu.sync_copy(zr, acc_sh.at[pl.ds(r0 + z * 8, 8)])
        if rpt % 8:
            pltpu.sync_copy(zr.at[pl.ds(0, rpt % 8)],
                            acc_sh.at[pl.ds(r0 + (rpt // 8) * 8, rpt % 8)])
        plsc.subcore_barrier()

        def half_blk(b2, k, stg, li, ssem):
            b = 2 * b2 + k
            base = sid * (e_total // _NS) + b * blk
            pltpu.sync_copy(src_h.at[pl.ds(base, blk)], si)
            pltpu.sync_copy(dst_h.at[pl.ds(base, blk)], di)
            pltpu.sync_copy(e_h.at[pl.ds(base, blk)], eb)
            cpy = pltpu.async_copy(xl_h.at[si], rows, sem)

            @pl.when(b2 > 0)
            def _():
                pltpu.make_async_copy(stg, acc_sh.at[li], ssem).wait()

            for g in range(ngrp):
                dstg = di[pl.ds(g * _LANES, _LANES)]
                eg = eb[pl.ds(g * _LANES, _LANES)]
                mg = plsc.load_gather(mb, [dstg])
                ex = jnp.exp(eg - mg)
                loc = dstg - nbase
                inr = (loc >= 0) & (loc < nh)
                wb[pl.ds(g * _LANES, _LANES)] = jnp.where(inr, ex, 0.0)
                li[pl.ds(g * _LANES, _LANES)] = jnp.clip(loc, 0, nhp - 1)
            cpy.wait()

            def edge_body(j, _):
                jv = jnp.full((_LANES,), j, jnp.int32)
                wv = plsc.load_gather(wb, [jv])
                for c in range(nch):
                    col = c * _LANES + lane
                    rowv = plsc.load_gather(rows, [jv, col])
                    plsc.store_scatter(stg, [jv, col], rowv * wv)
                plsc.store_scatter(stg, [jv, dout + lane],
                                   jnp.where(lane == 0, wv, 0.0))
                return 0

            lax.fori_loop(0, blk, edge_body, 0)
            pltpu.async_copy(stg, acc_sh.at[li], ssem, add=True)

        def blk_body(b2, _):
            half_blk(b2, 0, stg_a, li_a, sem_a)
            half_blk(b2, 1, stg_b, li_b, sem_b)
            return 0

        lax.fori_loop(0, nblk // 2, blk_body, 0)
        pltpu.make_async_copy(stg_a, acc_sh.at[li_a], sem_a).wait()
        pltpu.make_async_copy(stg_b, acc_sh.at[li_b], sem_b).wait()
        plsc.subcore_barrier()
        pltpu.sync_copy(acc_sh.at[pl.ds(sid * rpt, rpt)],
                        acc_h.at[cid, pl.ds(sid * rpt, rpt)])

    return kern(xl, src, dst, e, m)


def _gatv2_layer(x, src_p, dst_p, ea_p, loop_ea, e_real, p):
    n = x.shape[0]
    np_ = ((n + 16 * _NS - 1) // (16 * _NS)) * 16 * _NS
    xl = x @ p['Wl'] + p['bl']
    xr = x @ p['Wr'] + p['br']
    ep = ea_p @ p['We']
    lep = loop_ea @ p['We']

    e_edge, m2 = _edge_scores_sc(xl, xr, ep, src_p, dst_p, p['att'], np_)
    sf = xl + xr + lep
    sf = jnp.where(sf > 0, sf, 0.2 * sf)
    e_self = sf @ p['att']

    m_t = jnp.maximum(jnp.maximum(m2[0, :n], m2[1, :n]), e_self)
    ex_self = jnp.exp(e_self - m_t)

    src = src_p[:e_real]
    dst = dst_p[:e_real]
    ex = jnp.exp(e_edge[:e_real] - m_t[dst])
    den = jax.ops.segment_sum(ex, dst, num_segments=n) + ex_self
    num = jax.ops.segment_sum(ex[:, None] * xl[src], dst, num_segments=n) \
        + ex_self[:, None] * xl
    out = num / (den[:, None] + 1e-16)
    return out + p['bias']


def _ln_relu_body(x_ref, g_ref, b_ref, o_ref, *, relu):
    x = x_ref[...]
    mu = jnp.mean(x, axis=-1, keepdims=True)
    var = jnp.mean((x - mu) ** 2, axis=-1, keepdims=True)
    y = (x - mu) * jax.lax.rsqrt(var + 1e-5) * g_ref[...] + b_ref[...]
    if relu:
        y = jnp.maximum(y, 0.0)
    o_ref[...] = y


def _ln_relu(x, g, b, relu):
    n, d = x.shape
    blk = 1000
    return pl.pallas_call(
        functools.partial(_ln_relu_body, relu=relu),
        grid=(n // blk,),
        in_specs=[
            pl.BlockSpec((blk, d), lambda i: (i, 0)),
            pl.BlockSpec((d,), lambda i: (0,)),
            pl.BlockSpec((d,), lambda i: (0,)),
        ],
        out_specs=pl.BlockSpec((blk, d), lambda i: (i, 0)),
        out_shape=jax.ShapeDtypeStruct((n, d), x.dtype),
    )(x, g, b)


def kernel(x, edge_index, edge_features, params):
    src, dst = edge_index[0], edge_index[1]
    e_real = src.shape[0]
    n = x.shape[0]
    per_w = -(-e_real // (_NW * _BLK)) * _BLK
    e_pad = per_w * _NW
    pad = e_pad - e_real
    src_p = jnp.concatenate([src, jnp.zeros((pad,), src.dtype)])
    dst_p = jnp.concatenate([dst, jnp.full((pad,), n, dst.dtype)])
    ea_p = jnp.concatenate(
        [edge_features, jnp.zeros((pad, edge_features.shape[1]),
                                  edge_features.dtype)], axis=0)
    cnt = jax.ops.segment_sum(jnp.ones((e_real,), dtype=x.dtype), dst,
                              num_segments=n)
    loop_ea = jax.ops.segment_sum(edge_features, dst, num_segments=n) \
        / jnp.clip(cnt, 1.0)[:, None]

    out = _gatv2_layer(x, src_p, dst_p, ea_p, loop_ea, e_real, params['conv1'])
    out = _ln_relu(out, params['conv1']['g'], params['conv1']['b'], True)
    out = _gatv2_layer(out, src_p, dst_p, ea_p, loop_ea, e_real, params['conv2'])
    out = _ln_relu(out, params['conv2']['g'], params['conv2']['b'], True)
    out = _gatv2_layer(out, src_p, dst_p, ea_p, loop_ea, e_real, params['conv3'])
    out = _ln_relu(out, params['conv3']['g'], params['conv3']['b'], False)
    return out
